# Initial kernel scaffold; baseline (speedup 1.0000x reference)
import jax
import jax.numpy as jnp
from jax.experimental import pallas as pl


def kernel(x, edge_index, edge_attr, batch, W_type, W_chir, edge_emb1, edge_emb2, w1, b1, w2, b2, gamma, beta, pred_w, pred_b):
    raise NotImplementedError("write your pallas kernel here")


# speedup vs baseline: 3.5642x; 3.5642x over previous
import functools

import jax
import jax.numpy as jnp
import numpy as np
from jax import lax
from jax.experimental import pallas as pl
from jax.experimental.pallas import tpu as pltpu
from jax.experimental.pallas import tpu_sc as plsc

N = 10000
E = 160000
D = 256
H = 512
L = 5
G = 32
T = 12
HALF = 128
BM = 1000
GRID = N // BM
EP = 163840
ER = EP // 128
NC, NS = 2, 16
NP = N + 16
RW = 624
RT = N - NS * RW
J_SPMM = ER // NS // 8
J_CNT = ER // (NC * NS) // 8

@functools.cache
def _mesh():
  return plsc.VectorSubcoreMesh(
      core_axis_name="c", subcore_axis_name="s", num_cores=NC, num_subcores=NS)


def _spmm_sc(htab, base, srcadj2d, dst2d):
  @functools.partial(
      pl.kernel,
      out_type=jax.ShapeDtypeStruct((2 * N, HALF), jnp.float32),
      mesh=_mesh(),
      scratch_types=[
          pltpu.VMEM((8, 128), jnp.int32),
          pltpu.VMEM((8, 128), jnp.int32),
          pltpu.VMEM((128, HALF), jnp.float32),
          pltpu.VMEM_SHARED((NP, HALF), jnp.float32),
      ])
  def k(htab_hbm, base_hbm, src_hbm, dst_hbm, out_hbm, sidx, didx, rows, shared):
    c = lax.axis_index("c")
    s = lax.axis_index("s")
    pltpu.sync_copy(base_hbm.at[pl.ds(c * N + s * RW, RW)],
                    shared.at[pl.ds(s * RW, RW)])

    @pl.when(s == NS - 1)
    def _():
      pltpu.sync_copy(base_hbm.at[pl.ds(c * N + NS * RW, RT)],
                      shared.at[pl.ds(NS * RW, RT)])

    plsc.subcore_barrier()

    @pl.loop(0, J_SPMM)
    def _(jb):
      rb = s * (8 * J_SPMM) + jb * 8
      pltpu.sync_copy(src_hbm.at[pl.ds(c * ER + rb, 8)], sidx)
      pltpu.sync_copy(dst_hbm.at[pl.ds(rb, 8)], didx)
      for kk in range(8):
        pltpu.sync_copy(htab_hbm.at[sidx.at[kk]], rows)
        pltpu.sync_copy(rows, shared.at[didx.at[kk]], add=True)

    plsc.subcore_barrier()
    pltpu.sync_copy(shared.at[pl.ds(s * RW, RW)],
                    out_hbm.at[pl.ds(c * N + s * RW, RW)])

    @pl.when(s == NS - 1)
    def _():
      pltpu.sync_copy(shared.at[pl.ds(NS * RW, RT)],
                      out_hbm.at[pl.ds(c * N + NS * RW, RT)])

  return k(htab, base, srcadj2d, dst2d)


def _counts_sc(table, zeros_n16, comb2d, dst2d):
  @functools.partial(
      pl.kernel,
      out_type=jax.ShapeDtypeStruct((2 * N, 128), jnp.float32),
      mesh=_mesh(),
      scratch_types=[
          pltpu.VMEM((8, 128), jnp.int32),
          pltpu.VMEM((8, 128), jnp.int32),
          pltpu.VMEM((128, 128), jnp.float32),
          pltpu.VMEM_SHARED((NP, 128), jnp.float32),
      ])
  def k(tab_hbm, z_hbm, comb_hbm, dst_hbm, out_hbm, cidx, didx, rows, shared):
    c = lax.axis_index("c")
    s = lax.axis_index("s")
    pltpu.sync_copy(z_hbm.at[pl.ds(s * RW, RW)], shared.at[pl.ds(s * RW, RW)])

    @pl.when(s == NS - 1)
    def _():
      pltpu.sync_copy(z_hbm.at[pl.ds(NS * RW, RT)],
                      shared.at[pl.ds(NS * RW, RT)])

    plsc.subcore_barrier()

    @pl.loop(0, J_CNT)
    def _(jb):
      rb = (c * NS + s) * (8 * J_CNT) + jb * 8
      pltpu.sync_copy(comb_hbm.at[pl.ds(rb, 8)], cidx)
      pltpu.sync_copy(dst_hbm.at[pl.ds(rb, 8)], didx)
      for kk in range(8):
        pltpu.sync_copy(tab_hbm.at[cidx.at[kk]], rows)
        pltpu.sync_copy(rows, shared.at[didx.at[kk]], add=True)

    plsc.subcore_barrier()
    pltpu.sync_copy(shared.at[pl.ds(s * RW, RW)],
                    out_hbm.at[pl.ds(c * N + s * RW, RW)])

    @pl.when(s == NS - 1)
    def _():
      pltpu.sync_copy(shared.at[pl.ds(NS * RW, RT)],
                      out_hbm.at[pl.ds(c * N + NS * RW, RT)])

  return k(table, zeros_n16, comb2d, dst2d)


def _embed_tc(x0r, x1r, wt, wc):
  def body(x0_ref, x1_ref, wt_ref, wc_ref, o_ref):
    x0 = x0_ref[0]
    x1 = x1_ref[0]
    oh0 = (lax.broadcasted_iota(jnp.int32, (128, BM), 0) == x0).astype(jnp.float32)
    oh1 = (lax.broadcasted_iota(jnp.int32, (8, BM), 0) == x1).astype(jnp.float32)
    h = lax.dot_general(oh0, wt_ref[...], (((0,), (0,)), ((), ())),
                        preferred_element_type=jnp.float32, precision=lax.Precision.HIGHEST)
    h = h + lax.dot_general(oh1, wc_ref[...], (((0,), (0,)), ((), ())),
                            preferred_element_type=jnp.float32, precision=lax.Precision.HIGHEST)
    o_ref[0] = h[:, :HALF]
    o_ref[1] = h[:, HALF:]

  return pl.pallas_call(
      body,
      grid=(GRID,),
      in_specs=[
          pl.BlockSpec((1, 1, BM), lambda i: (i, 0, 0)),
          pl.BlockSpec((1, 1, BM), lambda i: (i, 0, 0)),
          pl.BlockSpec((128, D), lambda i: (0, 0)),
          pl.BlockSpec((8, D), lambda i: (0, 0)),
      ],
      out_specs=pl.BlockSpec((2, BM, HALF), lambda i: (0, i, 0)),
      out_shape=jax.ShapeDtypeStruct((2, N, HALF), jnp.float32),
  )(x0r, x1r, wt, wc)


def _comb_tc(ea0r, ea1r):
  def body(a_ref, b_ref, o_ref):
    o_ref[...] = a_ref[...] * 3 + b_ref[...]

  return pl.pallas_call(
      body,
      grid=(10,),
      in_specs=[pl.BlockSpec((128, 128), lambda i: (i, 0))] * 2,
      out_specs=pl.BlockSpec((128, 128), lambda i: (i, 0)),
      out_shape=jax.ShapeDtypeStruct((ER, 128), jnp.int32),
  )(ea0r, ea1r)


def _edge_contrib(p0, p1, et_ref):
  p = p0 + p1
  lane = lax.broadcasted_iota(jnp.int32, (1, 128), 1)
  p = jnp.where((lane == 9) | (lane == 10), 1.0, p)
  return jnp.dot(p, et_ref[...], preferred_element_type=jnp.float32, precision=lax.Precision.HIGHEST)


def _prep0_tc(h0, p2, et):
  def body(h_ref, p_ref, et_ref, base_ref):
    contrib = _edge_contrib(p_ref[0], p_ref[1], et_ref)
    base_ref[0] = h_ref[0] + contrib[:, :HALF]
    base_ref[1] = h_ref[1] + contrib[:, HALF:]

  return pl.pallas_call(
      body,
      grid=(GRID,),
      in_specs=[
          pl.BlockSpec((2, BM, HALF), lambda i: (0, i, 0)),
          pl.BlockSpec((2, BM, 128), lambda i: (0, i, 0)),
          pl.BlockSpec((128, D), lambda i: (0, 0)),
      ],
      out_specs=pl.BlockSpec((2, BM, HALF), lambda i: (0, i, 0)),
      out_shape=jax.ShapeDtypeStruct((2, N, HALF), jnp.float32),
  )(h0, p2, et)


def _bn_terms(st_ref, var_ref, g_ref, be_ref):
  mean = st_ref[0:1, :] / float(N)
  inv = 1.0 / jnp.sqrt(var_ref[0:1, :] + 1e-5)
  return mean, inv, g_ref[0:1, :], be_ref[0:1, :]


def _prep_tc(h2raw, stats, var, g_r, be_r, p2, et):
  def body(h_ref, st_ref, v_ref, g_ref, be_ref, p_ref, et_ref, hn_ref, base_ref):
    mean, inv, g, be = _bn_terms(st_ref, v_ref, g_ref, be_ref)
    contrib = _edge_contrib(p_ref[0], p_ref[1], et_ref)
    for c in range(2):
      sl = slice(c * HALF, (c + 1) * HALF)
      hn = (h_ref[c] - mean[:, sl]) * inv[:, sl] * g[:, sl] + be[:, sl]
      hn = jnp.maximum(hn, 0.0)
      hn_ref[c] = hn
      base_ref[c] = hn + contrib[:, sl]

  return pl.pallas_call(
      body,
      grid=(GRID,),
      in_specs=[
          pl.BlockSpec((2, BM, HALF), lambda i: (0, i, 0)),
          pl.BlockSpec((8, D), lambda i: (0, 0)),
          pl.BlockSpec((8, D), lambda i: (0, 0)),
          pl.BlockSpec((8, D), lambda i: (0, 0)),
          pl.BlockSpec((8, D), lambda i: (0, 0)),
          pl.BlockSpec((2, BM, 128), lambda i: (0, i, 0)),
          pl.BlockSpec((128, D), lambda i: (0, 0)),
      ],
      out_specs=[
          pl.BlockSpec((2, BM, HALF), lambda i: (0, i, 0)),
          pl.BlockSpec((2, BM, HALF), lambda i: (0, i, 0)),
      ],
      out_shape=[
          jax.ShapeDtypeStruct((2, N, HALF), jnp.float32),
          jax.ShapeDtypeStruct((2, N, HALF), jnp.float32),
      ],
  )(h2raw, stats, var, g_r, be_r, p2, et)


def _mlp_tc(agg, w1l, b1r, w2l, b2r):
  def body(a_ref, w1_ref, b1_ref, w2_ref, b2_ref, h2_ref, st_ref, acc):
    i = pl.program_id(0)
    b16 = jnp.bfloat16
    a = jnp.concatenate([a_ref[0], a_ref[1]], axis=1).astype(b16)
    mid = jnp.dot(a, w1_ref[...].astype(b16), preferred_element_type=jnp.float32)
    mid = jnp.maximum(mid + b1_ref[0:1, :], 0.0)
    h2 = jnp.dot(mid.astype(b16), w2_ref[...].astype(b16),
                 preferred_element_type=jnp.float32)
    h2 = h2 + b2_ref[0:1, :]
    h2_ref[0] = h2[:, :HALF]
    h2_ref[1] = h2[:, HALF:]

    @pl.when(i == 0)
    def _():
      acc[...] = jnp.zeros_like(acc)

    acc[0:1, :] += jnp.sum(h2, axis=0, keepdims=True)
    st_ref[...] = acc[...]

  return pl.pallas_call(
      body,
      grid=(GRID,),
      in_specs=[
          pl.BlockSpec((2, BM, HALF), lambda i: (0, i, 0)),
          pl.BlockSpec((D, H), lambda i: (0, 0)),
          pl.BlockSpec((8, H), lambda i: (0, 0)),
          pl.BlockSpec((H, D), lambda i: (0, 0)),
          pl.BlockSpec((8, D), lambda i: (0, 0)),
      ],
      out_specs=[
          pl.BlockSpec((2, BM, HALF), lambda i: (0, i, 0)),
          pl.BlockSpec((8, D), lambda i: (0, 0)),
      ],
      out_shape=[
          jax.ShapeDtypeStruct((2, N, HALF), jnp.float32),
          jax.ShapeDtypeStruct((8, D), jnp.float32),
      ],
      scratch_shapes=[pltpu.VMEM((8, D), jnp.float32)],
  )(agg, w1l, b1r, w2l, b2r)


def _var_tc(h2raw, stats):
  def body(h_ref, st_ref, v_ref, acc):
    i = pl.program_id(0)
    mean = st_ref[0:1, :] / float(N)
    h2 = jnp.concatenate([h_ref[0], h_ref[1]], axis=1)
    d = h2 - mean

    @pl.when(i == 0)
    def _():
      acc[...] = jnp.zeros_like(acc)

    acc[0:1, :] += jnp.sum(d * d, axis=0, keepdims=True)
    v_ref[...] = acc[...] / float(N)

  return pl.pallas_call(
      body,
      grid=(GRID,),
      in_specs=[
          pl.BlockSpec((2, BM, HALF), lambda i: (0, i, 0)),
          pl.BlockSpec((8, D), lambda i: (0, 0)),
      ],
      out_specs=pl.BlockSpec((8, D), lambda i: (0, 0)),
      out_shape=jax.ShapeDtypeStruct((8, D), jnp.float32),
      scratch_shapes=[pltpu.VMEM((8, D), jnp.float32)],
  )(h2raw, stats)


def _final_tc(h2raw, stats, var, g_r, be_r, batch3, pw_pad, pb_pad):
  def body(h_ref, st_ref, v_ref, g_ref, be_ref, b_ref, pw_ref, pb_ref, o_ref,
           gacc, cacc):
    i = pl.program_id(0)
    mean, inv, g, be = _bn_terms(st_ref, v_ref, g_ref, be_ref)
    hfull = jnp.concatenate([h_ref[0], h_ref[1]], axis=1)
    hfull = (hfull - mean) * inv * g + be
    bb = b_ref[0]
    oh = (lax.broadcasted_iota(jnp.int32, (G, BM), 0) == bb).astype(jnp.float32)

    @pl.when(i == 0)
    def _():
      gacc[...] = jnp.zeros_like(gacc)
      cacc[...] = jnp.zeros_like(cacc)

    gacc[...] += jnp.dot(oh, hfull, preferred_element_type=jnp.float32, precision=lax.Precision.HIGHEST)
    cacc[...] += jnp.dot(oh, jnp.ones((BM, 8), jnp.float32),
                         preferred_element_type=jnp.float32, precision=lax.Precision.HIGHEST)
    rep = gacc[...] / jnp.maximum(cacc[:, 0:1], 1.0)
    o_ref[...] = jnp.dot(rep.astype(jnp.bfloat16),
                         pw_ref[...].astype(jnp.bfloat16),
                         preferred_element_type=jnp.float32) + pb_ref[0:1, :]

  return pl.pallas_call(
      body,
      grid=(GRID,),
      in_specs=[
          pl.BlockSpec((2, BM, HALF), lambda i: (0, i, 0)),
          pl.BlockSpec((8, D), lambda i: (0, 0)),
          pl.BlockSpec((8, D), lambda i: (0, 0)),
          pl.BlockSpec((8, D), lambda i: (0, 0)),
          pl.BlockSpec((8, D), lambda i: (0, 0)),
          pl.BlockSpec((1, 1, BM), lambda i: (i, 0, 0)),
          pl.BlockSpec((D, 128), lambda i: (0, 0)),
          pl.BlockSpec((8, 128), lambda i: (0, 0)),
      ],
      out_specs=pl.BlockSpec((G, 128), lambda i: (0, 0)),
      out_shape=jax.ShapeDtypeStruct((G, 128), jnp.float32),
      scratch_shapes=[
          pltpu.VMEM((G, D), jnp.float32),
          pltpu.VMEM((G, 8), jnp.float32),
      ],
  )(h2raw, stats, var, g_r, be_r, batch3, pw_pad, pb_pad)


def _row_pad(v, rows=8):
  return jnp.pad(v.reshape(1, -1), ((0, rows - 1), (0, 0)))


def kernel(x, edge_index, edge_attr, batch, W_type, W_chir, edge_emb1,
           edge_emb2, w1, b1, w2, b2, gamma, beta, pred_w, pred_b):
  xi = x.astype(jnp.int32)
  ei = edge_index.astype(jnp.int32)
  ea = edge_attr.astype(jnp.int32)
  bt = batch.astype(jnp.int32)

  x0r = xi[:, 0].reshape(GRID, 1, BM)
  x1r = xi[:, 1].reshape(GRID, 1, BM)
  batch3 = bt.reshape(GRID, 1, BM)

  pad = EP - E
  src = jnp.concatenate([ei[0], jnp.zeros((pad,), jnp.int32)])
  dst = jnp.concatenate([ei[1], jnp.full((pad,), N, jnp.int32)])
  srcadj2d = jnp.concatenate([src, src + N]).reshape(2 * ER, 128)
  dst2d = dst.reshape(ER, 128)
  ea0r = jnp.concatenate([ea[:, 0], jnp.zeros((pad,), jnp.int32)]).reshape(ER, 128)
  ea1r = jnp.concatenate([ea[:, 1], jnp.zeros((pad,), jnp.int32)]).reshape(ER, 128)

  wt = jnp.pad(W_type, ((0, 128 - W_type.shape[0]), (0, 0)))
  wc = jnp.pad(W_chir, ((0, 8 - W_chir.shape[0]), (0, 0)))

  tnp = np.zeros((24, 128), np.float32)
  for k in range(18):
    tnp[k, k // 3] = 1.0
    tnp[k, 6 + k % 3] = 1.0
  tonehot = jnp.asarray(tnp)
  zeros_n16 = jnp.zeros((N, 128), jnp.float32)

  comb2d = _comb_tc(ea0r, ea1r)
  p2 = _counts_sc(tonehot, zeros_n16, comb2d, dst2d).reshape(2, N, 128)
  h0 = _embed_tc(x0r, x1r, wt, wc)

  ets = [
      jnp.concatenate([
          edge_emb1[l], edge_emb2[l], edge_emb1[l, 4:5], edge_emb2[l, 0:1],
          jnp.zeros((117, D), jnp.float32)
      ], axis=0) for l in range(L)
  ]

  h2raw, stats, var = None, None, None
  for l in range(L):
    if l == 0:
      base = _prep0_tc(h0, p2, ets[0])
      tab = h0.reshape(2 * N, HALF)
    else:
      hn, base = _prep_tc(h2raw, stats, var, _row_pad(gamma[l - 1]),
                          _row_pad(beta[l - 1]), p2, ets[l])
      tab = hn.reshape(2 * N, HALF)
    agg = _spmm_sc(tab, base.reshape(2 * N, HALF), srcadj2d, dst2d)
    h2raw, stats = _mlp_tc(agg.reshape(2, N, HALF), w1[l], _row_pad(b1[l]),
                           w2[l], _row_pad(b2[l]))
    var = _var_tc(h2raw, stats)

  pw_pad = jnp.pad(pred_w, ((0, 0), (0, 128 - T)))
  pb_pad = jnp.pad(pred_b.reshape(1, T), ((0, 7), (0, 128 - T)))
  out = _final_tc(h2raw, stats, var, _row_pad(gamma[L - 1]),
                  _row_pad(beta[L - 1]), batch3, pw_pad, pb_pad)
  return out[:, :T]
